# Initial kernel scaffold; baseline (speedup 1.0000x reference)
import jax
import jax.numpy as jnp
from jax.experimental import pallas as pl


def kernel(vgids, vsids, vcids, vgprices, goods_table, shop_table, cate_table, price_table):
    raise NotImplementedError("write your pallas kernel here")


# speedup vs baseline: 10.6802x; 10.6802x over previous
import functools

import jax
import jax.numpy as jnp
from jax import lax
from jax.experimental import pallas as pl
from jax.experimental.pallas import tpu as pltpu
import jax.experimental.pallas.tpu_sc as plsc

_B, _L, _D = 4096, 50, 64
_NC, _NS = 2, 16
_NW = _NC * _NS
_BPW = _B // _NW
_PPW = _BPW * _L
_CHUNK = 128
_NCHUNK = _PPW // _CHUNK


def _body(gidx, sidx, cidx, pidx, dst, zeros,
          goods, shop, cate, price, out,
          idxv, dstv, rows, acc, sem_g, sem_s):
    sid = lax.axis_index("s")
    wid = sid * _NC + lax.axis_index("c")
    base = wid * _BPW
    pltpu.sync_copy(dst.at[sid], dstv)
    for f, (idx_hbm, table) in enumerate(
        ((gidx, goods), (sidx, shop), (cidx, cate), (pidx, price))):
        pltpu.sync_copy(zeros, acc.at[pl.ds(sid * _BPW, _BPW)])
        pltpu.sync_copy(idx_hbm.at[wid], idxv)

        def step(j, carry):
            pltpu.async_copy(table.at[idxv.at[j]], rows, sem_g).wait()
            pltpu.async_copy(rows, acc.at[dstv.at[j]], sem_s, add=True).wait()
            return carry

        lax.fori_loop(0, _NCHUNK, step, 0)
        pltpu.sync_copy(acc.at[pl.ds(sid * _BPW, _BPW)],
                        out.at[pl.ds(base, _BPW), pl.ds(f * _D, _D)])


@jax.jit
def kernel(vgids, vsids, vcids, vgprices,
           goods_table, shop_table, cate_table, price_table):
    shape3 = (_NW, _NCHUNK, _CHUNK)
    gidx = vgids.astype(jnp.int32).reshape(shape3)
    sidx = vsids.astype(jnp.int32).reshape(shape3)
    cidx = vcids.astype(jnp.int32).reshape(shape3)
    pidx = vgprices.astype(jnp.int32).reshape(shape3)
    local = (jnp.arange(_PPW, dtype=jnp.int32) // _L).reshape(_NCHUNK, _CHUNK)
    dst = (jnp.arange(_NS, dtype=jnp.int32)[:, None, None] * _BPW
           + local[None]).astype(jnp.int32)
    zeros = jnp.zeros((_BPW, _D), jnp.float32)

    run = pl.kernel(
        _body,
        out_type=jax.ShapeDtypeStruct((_B, 4 * _D), jnp.float32),
        mesh=plsc.VectorSubcoreMesh(core_axis_name="c", subcore_axis_name="s"),
        compiler_params=pltpu.CompilerParams(use_tc_tiling_on_sc=False),
        scratch_types=[
            pltpu.VMEM((_NCHUNK, _CHUNK), jnp.int32),
            pltpu.VMEM((_NCHUNK, _CHUNK), jnp.int32),
            pltpu.VMEM((_CHUNK, _D), jnp.float32),
            pltpu.VMEM_SHARED((_NS * _BPW, _D), jnp.float32),
            pltpu.SemaphoreType.DMA,
            pltpu.SemaphoreType.DMA,
        ],
    )
    return run(gidx, sidx, cidx, pidx, dst, zeros,
               goods_table, shop_table, cate_table, price_table)


# speedup vs baseline: 14.8163x; 1.3873x over previous
import functools

import jax
import jax.numpy as jnp
from jax import lax
from jax.experimental import pallas as pl
from jax.experimental.pallas import tpu as pltpu
import jax.experimental.pallas.tpu_sc as plsc

_B, _L, _D = 4096, 50, 64
_NC, _NS = 2, 16
_NW = _NC * _NS
_BPW = _B // _NW
_PPW = _BPW * _L
_CHUNK = 128
_NCHUNK = _PPW // _CHUNK
_NBUF = 4


def _body(idx_all, dst, zeros, goods, shop, cate, price, out,
          idxv, dstv, rows, acc0, acc1, acc2, acc3,
          gsem, ssem, zsem):
    sid = lax.axis_index("s")
    wid = sid * _NC + lax.axis_index("c")
    base = wid * _BPW
    accs = (acc0, acc1, acc2, acc3)
    tables = (goods, shop, cate, price)

    pltpu.sync_copy(idx_all.at[wid], idxv)
    pltpu.sync_copy(dst.at[sid], dstv)
    my = pl.ds(sid * _BPW, _BPW)
    for f in range(4):
        pltpu.async_copy(zeros, accs[f].at[my], zsem)
    for f in range(4):
        pltpu.make_async_copy(zeros, accs[f].at[my], zsem).wait()

    pending = [False] * _NBUF

    def gather(f, j, slot):
        pltpu.async_copy(tables[f].at[idxv.at[f, j]], rows.at[slot],
                         gsem.at[slot])

    def wait_gather(f, slot):
        pltpu.make_async_copy(tables[f].at[idxv.at[f, 0]], rows.at[slot],
                              gsem.at[slot]).wait()

    def scatter(f, j, slot):
        pltpu.async_copy(rows.at[slot], accs[f].at[dstv.at[j]],
                         ssem.at[slot], add=True)

    def wait_scatter(f, slot):
        pltpu.make_async_copy(rows.at[slot], accs[f].at[dstv.at[0]],
                              ssem.at[slot]).wait()

    for f in range(4):
        for b in range(_NBUF):
            if pending[b]:
                wait_scatter(f - 1, b)
                pending[b] = False
            gather(f, b, b)
        wait_gather(f, 0)
        scatter(f, 0, 0)

        def step(j, carry):
            p = j % _NBUF
            p1 = (j - 1) % _NBUF
            wait_gather(f, p)
            scatter(f, j, p)
            wait_scatter(f, p1)
            gather(f, j - 1 + _NBUF, p1)
            return carry

        lax.fori_loop(1, _NCHUNK - _NBUF + 1, step, 0, unroll=2)

        for j in range(_NCHUNK - _NBUF + 1, _NCHUNK):
            p = j % _NBUF
            wait_gather(f, p)
            scatter(f, j, p)
        for j in range(_NCHUNK - _NBUF, _NCHUNK):
            pending[j % _NBUF] = True

    for b in range(_NBUF):
        if pending[b]:
            wait_scatter(3, b)
            pending[b] = False
    for f in range(4):
        pltpu.sync_copy(accs[f].at[my],
                        out.at[pl.ds(base, _BPW), pl.ds(f * _D, _D)])


@jax.jit
def kernel(vgids, vsids, vcids, vgprices,
           goods_table, shop_table, cate_table, price_table):
    shape3 = (_NW, _NCHUNK, _CHUNK)
    idx_all = jnp.stack(
        [vgids.astype(jnp.int32).reshape(shape3),
         vsids.astype(jnp.int32).reshape(shape3),
         vcids.astype(jnp.int32).reshape(shape3),
         vgprices.astype(jnp.int32).reshape(shape3)], axis=1)
    local = (jnp.arange(_PPW, dtype=jnp.int32) // _L).reshape(_NCHUNK, _CHUNK)
    dst = (jnp.arange(_NS, dtype=jnp.int32)[:, None, None] * _BPW
           + local[None]).astype(jnp.int32)
    zeros = jnp.zeros((_BPW, _D), jnp.float32)

    acc_t = pltpu.VMEM_SHARED((_NS * _BPW, _D), jnp.float32)
    run = pl.kernel(
        _body,
        out_type=jax.ShapeDtypeStruct((_B, 4 * _D), jnp.float32),
        mesh=plsc.VectorSubcoreMesh(core_axis_name="c", subcore_axis_name="s"),
        compiler_params=pltpu.CompilerParams(use_tc_tiling_on_sc=False),
        scratch_types=[
            pltpu.VMEM((4, _NCHUNK, _CHUNK), jnp.int32),
            pltpu.VMEM((_NCHUNK, _CHUNK), jnp.int32),
            pltpu.VMEM((_NBUF, _CHUNK, _D), jnp.float32),
            acc_t, acc_t, acc_t, acc_t,
            pltpu.SemaphoreType.DMA((_NBUF,)),
            pltpu.SemaphoreType.DMA((_NBUF,)),
            pltpu.SemaphoreType.DMA,
        ],
    )
    return run(idx_all, dst, zeros,
               goods_table, shop_table, cate_table, price_table)
